# baseline (speedup 1.0000x reference)
import jax
import jax.numpy as jnp
from jax import lax
from jax.experimental import pallas as pl
from jax.experimental.pallas import tpu as pltpu
from jax.experimental.pallas import tpu_sc as plsc

_WIDTH = 2048
_HEIGHT = 2048
_N_UVS = 1048576
_NTEX = _HEIGHT * _WIDTH
_NELEM = _NTEX * 4
_HALF = _NELEM
_NWIN = _NELEM // 16

_NC = 2
_NS = 16
_NW = _NC * _NS
_L = 16

_SZE = _NELEM // _NW
_BB = 16384
_TB = _BB // 4
_TSTAGE = _TB + 16
_BSTEPS = _SZE // _BB
_NBUF = 2
_IVLEN = _BB + 64

_CHUNK = 128
_SCH = 1024
_QS = _SCH // _CHUNK
_PER_W = _N_UVS // _NW
_ITERS = _PER_W // _SCH


def _build_body(p0_hbm, p1_hbm, p2_hbm, u_hbm, v_hbm,
                tbl_hbm, rA_hbm, rB_hbm, cb_hbm, cbd_hbm, a_hbm, b_hbm,
                *scratch):
    wid = lax.axis_index("s") * _NC + lax.axis_index("c")
    ebase = wid * _SZE
    ubase = wid * _PER_W
    lane = lax.iota(jnp.int32, _L)
    lane4 = lane * 4
    planes = (p0_hbm, p1_hbm, p2_hbm)
    stages = tuple(tuple(scratch[3 * k:3 * k + 3]) for k in range(_NBUF))
    n = 3 * _NBUF
    ivs = tuple(scratch[n:n + _NBUF]); n += _NBUF
    in_sems = tuple(scratch[n:n + _NBUF]); n += _NBUF
    out_sems = tuple(scratch[n:n + _NBUF]); n += _NBUF
    uv_bufs = tuple(tuple(scratch[n + 2 * k:n + 2 * k + 2])
                    for k in range(2)); n += 4
    ix_bufs = tuple(tuple(scratch[n + 6 * k:n + 6 * k + 6])
                    for k in range(2)); n += 12
    uvin_sems = tuple(scratch[n:n + 2]); n += 2
    ixout_sems = tuple(scratch[n:n + 2]); n += 2
    ix_hbms = (rA_hbm, rB_hbm, cb_hbm, cbd_hbm, a_hbm, b_hbm)

    def fire_uv(j, k):
        off = pl.multiple_of(ubase + j * _SCH, _SCH)
        pltpu.async_copy(u_hbm.at[pl.ds(off, _SCH)], uv_bufs[k][0],
                         uvin_sems[k])
        pltpu.async_copy(v_hbm.at[pl.ds(off, _SCH)], uv_bufs[k][1],
                         uvin_sems[k])

    def wait_uv(k):
        for c in range(2):
            pltpu.make_async_copy(u_hbm.at[pl.ds(0, _SCH)],
                                  uv_bufs[k][c], uvin_sems[k]).wait()

    def index_phase(k):
        def body(i2, carry):
            for t in range(2):
                i = i2 * 2 + t
                sl = pl.ds(pl.multiple_of(_L * i, _L), _L)
                us = uv_bufs[k][0][sl]
                vs = uv_bufs[k][1][sl]
                u = ((us + 1.0) * 0.5) * (_WIDTH - 1)
                v = ((vs + 1.0) * 0.5) * (_HEIGHT - 1)
                u0 = u.astype(jnp.int32)
                v0 = v.astype(jnp.int32)
                af = u - u0.astype(jnp.float32)
                bf = v - v0.astype(jnp.float32)
                u1 = u0 + jnp.where(af > 0.0, 1, 0)
                dv = jnp.where(bf > 0.0, 1, 0)
                s00 = u0 * _WIDTH + v0
                s10 = u1 * _WIDTH + v0
                w0 = s00 >> 1
                w1 = s10 >> 1
                cb = (s00 & 1) * 4
                ix_bufs[k][0][sl] = (w0 >> 1) + ((w0 & 1) << 20)
                ix_bufs[k][1][sl] = (w1 >> 1) + ((w1 & 1) << 20)
                ix_bufs[k][2][sl] = cb
                ix_bufs[k][3][sl] = cb + dv * 4
                ix_bufs[k][4][sl] = af
                ix_bufs[k][5][sl] = bf
            return carry

        lax.fori_loop(0, _SCH // _L // 2, body, 0)

    def fire_ixout(j, k):
        off = pl.multiple_of(ubase + j * _SCH, _SCH)
        for c in range(6):
            pltpu.async_copy(ix_bufs[k][c], ix_hbms[c].at[pl.ds(off, _SCH)],
                             ixout_sems[k])

    def wait_ixout(k):
        for c in range(6):
            pltpu.make_async_copy(ix_bufs[k][c],
                                  ix_hbms[c].at[pl.ds(0, _SCH)],
                                  ixout_sems[k]).wait()

    def fire_stage(j, k):
        t0 = pl.multiple_of((ebase + j * _BB) >> 2, 8)
        for c in range(3):
            pltpu.async_copy(planes[c].at[pl.ds(t0, _TSTAGE)],
                             stages[k][c], in_sems[k])

    def wait_stage(k):
        for c in range(3):
            pltpu.make_async_copy(planes[c].at[pl.ds(0, _TSTAGE)],
                                  stages[k][c], in_sems[k]).wait()

    def fire_out(j, k):
        eb = pl.multiple_of(ebase + j * _BB, 8)
        pltpu.async_copy(ivs[k].at[pl.ds(0, _BB)],
                         tbl_hbm.at[pl.ds(eb, _BB)], out_sems[k])
        pltpu.async_copy(ivs[k].at[pl.ds(8, _BB)],
                         tbl_hbm.at[pl.ds(_HALF + eb, _BB)], out_sems[k])

    def wait_out(k):
        for _ in range(2):
            pltpu.make_async_copy(ivs[k].at[pl.ds(0, _BB)],
                                  tbl_hbm.at[pl.ds(0, _BB)],
                                  out_sems[k]).wait()

    def interleave(k):
        for g in range(_TSTAGE // _L):
            gb = 64 * g
            for c in range(3):
                val = stages[k][c][pl.ds(_L * g, _L)]
                plsc.store_scatter(ivs[k], [lane4 + (gb + c)], val)

    for p in range(_NBUF - 1):
        fire_stage(p, p)
    fire_uv(0, 0)

    def superstep(ss, carry):
        for k in range(_NBUF):
            j = ss * _NBUF + k
            kn = (k + _NBUF - 1) % _NBUF
            if k == 0:
                fire_stage(j + _NBUF - 1, kn)
                fire_uv(j + 1, 1 - k)
            else:
                @pl.when(ss < _BSTEPS // _NBUF - 1)
                def _():
                    fire_stage(j + _NBUF - 1, kn)
                    fire_uv(j + 1, 1 - k)
            wait_stage(k)

            @pl.when(ss >= 1)
            def _():
                wait_out(k)

            interleave(k)
            fire_out(j, k)

            wait_uv(k)

            @pl.when(ss >= 1)
            def _():
                wait_ixout(k)

            index_phase(k)
            fire_ixout(j, k)
        return carry

    lax.fori_loop(0, _BSTEPS // _NBUF, superstep, 0)
    for k in range(_NBUF):
        wait_out(k)
        wait_ixout(k)


def _gather_body(rA_hbm, rB_hbm, cb_hbm, cbd_hbm, a_hbm, b_hbm, tbl_hbm,
                 ro_hbm, go_hbm, bo_hbm,
                 rA_a, rB_a, rA_b, rB_b,
                 cb_a, cbd_a, cb_b, cbd_b, a_a, b_a, a_b, b_b,
                 cA_a, cB_a, cA_b, cB_b,
                 or_a, og_a, ob_a, or_b, og_b, ob_b,
                 in_a, in_b, g_a, g_b, out_a, out_b):
    wid = lax.axis_index("s") * _NC + lax.axis_index("c")
    base = wid * _PER_W
    lane = lax.iota(jnp.int32, _L)
    r_bufs = ((rA_a, rB_a), (rA_b, rB_b))
    col_bufs = ((cb_a, cbd_a), (cb_b, cbd_b))
    w_bufs = ((a_a, b_a), (a_b, b_b))
    c_bufs = ((cA_a, cB_a), (cA_b, cB_b))
    o_bufs = ((or_a, og_a, ob_a), (or_b, og_b, ob_b))
    in_sems = (in_a, in_b)
    g_sems = (g_a, g_b)
    out_sems = (out_a, out_b)
    out_hbms = (ro_hbm, go_hbm, bo_hbm)
    ix_hbms = (rA_hbm, rB_hbm, cb_hbm, cbd_hbm, a_hbm, b_hbm)

    def ix_buf(k, c):
        return (r_bufs[k] + col_bufs[k] + w_bufs[k])[c]

    def fire_in(off, k):
        for c in range(6):
            pltpu.async_copy(ix_hbms[c].at[pl.ds(off, _SCH)], ix_buf(k, c),
                             in_sems[k])

    def wait_in(k):
        for c in range(6):
            pltpu.make_async_copy(ix_hbms[c].at[pl.ds(0, _SCH)],
                                  ix_buf(k, c), in_sems[k]).wait()

    def fire_gathers(k):
        for c in range(2):
            for q in range(_QS):
                pltpu.async_copy(
                    tbl_hbm.at[r_bufs[k][c].at[pl.ds(_CHUNK * q, _CHUNK)]],
                    c_bufs[k][c].at[pl.ds(_CHUNK * q, _CHUNK)], g_sems[k])

    def wait_gathers(k):
        for c in range(2):
            for q in range(_QS):
                pltpu.make_async_copy(
                    tbl_hbm.at[r_bufs[k][0].at[pl.ds(0, _CHUNK)]],
                    c_bufs[k][c].at[pl.ds(_CHUNK * q, _CHUNK)],
                    g_sems[k]).wait()

    def combine(k):
        cA_v, cB_v = c_bufs[k]

        def body(i2, carry):
            for t in range(2):
                i = i2 * 2 + t
                sl = pl.ds(pl.multiple_of(_L * i, _L), _L)
                af = w_bufs[k][0][sl]
                bf = w_bufs[k][1][sl]
                cb = col_bufs[k][0][sl]
                cbd = col_bufs[k][1][sl]
                naf = 1.0 - af
                nbf = 1.0 - bf
                row16 = lane + (_L * i)
                for ch in range(3):
                    c00 = plsc.load_gather(cA_v, [row16, cb + ch])
                    c01 = plsc.load_gather(cA_v, [row16, cbd + ch])
                    c10 = plsc.load_gather(cB_v, [row16, cb + ch])
                    c11 = plsc.load_gather(cB_v, [row16, cbd + ch])
                    x = ((c00 * af + c10 * naf) * bf
                         + (c01 * af + c11 * naf) * nbf)
                    xc = jnp.minimum(jnp.maximum(x, -9.0), 9.0)
                    e = jnp.exp(xc + xc)
                    tt = (e - 1.0) / (e + 1.0)
                    o_bufs[k][ch][sl] = tt
            return carry

        lax.fori_loop(0, _SCH // _L // 2, body, 0)

    def fire_outs(off, k):
        for ch in range(3):
            pltpu.async_copy(o_bufs[k][ch],
                             out_hbms[ch].at[pl.ds(off, _SCH)], out_sems[k])

    def wait_outs(k):
        for ch in range(3):
            pltpu.make_async_copy(o_bufs[k][ch],
                                  out_hbms[ch].at[pl.ds(0, _SCH)],
                                  out_sems[k]).wait()

    fire_in(pl.multiple_of(base, _SCH), 0)

    def superstep(ss, carry):
        for k in range(2):
            j = ss * 2 + k
            off = pl.multiple_of(base + j * _SCH, _SCH)
            wait_in(k)
            fire_gathers(k)

            if k == 0:
                @pl.when(ss >= 1)
                def _():
                    wait_gathers(1)

                    @pl.when(ss >= 2)
                    def _():
                        wait_outs(1)

                    combine(1)
                    fire_outs(off - _SCH, 1)
                fire_in(off + _SCH, 1)
            else:
                wait_gathers(0)

                @pl.when(ss >= 1)
                def _():
                    wait_outs(0)

                combine(0)
                fire_outs(off - _SCH, 0)

                @pl.when(ss < _ITERS // 2 - 1)
                def _():
                    fire_in(off + _SCH, 0)
        return carry

    lax.fori_loop(0, _ITERS // 2, superstep, 0)

    last = pl.multiple_of(base + (_ITERS - 1) * _SCH, _SCH)
    wait_gathers(1)
    wait_outs(1)
    combine(1)
    fire_outs(last, 1)
    wait_outs(0)
    wait_outs(1)


_MESH = dict(core_axis_name="c", subcore_axis_name="s",
             num_cores=_NC, num_subcores=_NS)
_CPARAMS = pltpu.CompilerParams(
    needs_layout_passes=False, use_tc_tiling_on_sc=False)


def kernel(uvs, texture):
    u = uvs[:, 0]
    v = uvs[:, 1]
    planes = [jnp.pad(texture[:, :, c].reshape(-1), (0, 16))
              for c in range(3)]

    build = pl.kernel(
        _build_body,
        out_type=(jax.ShapeDtypeStruct((2 * _HALF,), jnp.float32),
                  jax.ShapeDtypeStruct((_N_UVS,), jnp.int32),
                  jax.ShapeDtypeStruct((_N_UVS,), jnp.int32),
                  jax.ShapeDtypeStruct((_N_UVS,), jnp.int32),
                  jax.ShapeDtypeStruct((_N_UVS,), jnp.int32),
                  jax.ShapeDtypeStruct((_N_UVS,), jnp.float32),
                  jax.ShapeDtypeStruct((_N_UVS,), jnp.float32)),
        mesh=plsc.VectorSubcoreMesh(**_MESH),
        scratch_types=(
            [pltpu.VMEM((_TSTAGE,), jnp.float32)] * (3 * _NBUF)
            + [pltpu.VMEM((_IVLEN,), jnp.float32)] * _NBUF
            + [pltpu.SemaphoreType.DMA] * (2 * _NBUF)
            + [pltpu.VMEM((_SCH,), jnp.float32)] * 4
            + ([pltpu.VMEM((_SCH,), jnp.int32)] * 4
               + [pltpu.VMEM((_SCH,), jnp.float32)] * 2) * 2
            + [pltpu.SemaphoreType.DMA] * 4),
        compiler_params=_CPARAMS,
    )
    tbl_flat, rA, rB, cbv, cbdv, av, bv = build(*planes, u, v)
    tbl = tbl_flat.reshape(2 * _NWIN, 16)

    gather = pl.kernel(
        _gather_body,
        out_type=(jax.ShapeDtypeStruct((_N_UVS,), jnp.float32),) * 3,
        mesh=plsc.VectorSubcoreMesh(**_MESH),
        scratch_types=(
            [pltpu.VMEM((_SCH,), jnp.int32)] * 4
            + [pltpu.VMEM((_SCH,), jnp.int32)] * 4
            + [pltpu.VMEM((_SCH,), jnp.float32)] * 4
            + [pltpu.VMEM((_SCH, 16), jnp.float32)] * 4
            + [pltpu.VMEM((_SCH,), jnp.float32)] * 6
            + [pltpu.SemaphoreType.DMA] * 6),
        compiler_params=_CPARAMS,
    )
    r, g, b = gather(rA, rB, cbv, cbdv, av, bv, tbl)
    return jnp.stack([r, g, b], axis=1)


# speedup vs baseline: 1.0793x; 1.0793x over previous
import jax
import jax.numpy as jnp
from jax import lax
from jax.experimental import pallas as pl
from jax.experimental.pallas import tpu as pltpu
from jax.experimental.pallas import tpu_sc as plsc

_WIDTH = 2048
_HEIGHT = 2048
_N_UVS = 1048576
_NTEX = _HEIGHT * _WIDTH
_NELEM = _NTEX * 4
_HALF = _NELEM
_NWIN = _NELEM // 16

_NC = 2
_NS = 16
_NW = _NC * _NS
_L = 16

_SZE = _NELEM // _NW
_BB = 16384
_TB = _BB // 4
_TSTAGE = _TB + 16
_BSTEPS = _SZE // _BB
_NBUF = 2
_IVLEN = _BB + 64

_CHUNK = 128
_SCH = 1024
_QS = _SCH // _CHUNK
_PER_W = _N_UVS // _NW
_ITERS = _PER_W // _SCH


def _build_body(p0_hbm, p1_hbm, p2_hbm, tbl_hbm, *scratch):
    wid = lax.axis_index("s") * _NC + lax.axis_index("c")
    ebase = wid * _SZE
    lane = lax.iota(jnp.int32, _L)
    lane4 = lane * 4
    planes = (p0_hbm, p1_hbm, p2_hbm)
    stages = tuple(tuple(scratch[3 * k:3 * k + 3]) for k in range(_NBUF))
    ivs = tuple(scratch[3 * _NBUF:4 * _NBUF])
    in_sems = tuple(scratch[4 * _NBUF:5 * _NBUF])
    out_sems = tuple(scratch[5 * _NBUF:6 * _NBUF])

    def fire_stage(j, k):
        t0 = pl.multiple_of((ebase + j * _BB) >> 2, 8)
        for c in range(3):
            pltpu.async_copy(planes[c].at[pl.ds(t0, _TSTAGE)],
                             stages[k][c], in_sems[k])

    def wait_stage(k):
        for c in range(3):
            pltpu.make_async_copy(planes[c].at[pl.ds(0, _TSTAGE)],
                                  stages[k][c], in_sems[k]).wait()

    def fire_out(j, k):
        eb = pl.multiple_of(ebase + j * _BB, 8)
        pltpu.async_copy(ivs[k].at[pl.ds(0, _BB)],
                         tbl_hbm.at[pl.ds(eb, _BB)], out_sems[k])
        pltpu.async_copy(ivs[k].at[pl.ds(8, _BB)],
                         tbl_hbm.at[pl.ds(_HALF + eb, _BB)], out_sems[k])

    def wait_out(k):
        for _ in range(2):
            pltpu.make_async_copy(ivs[k].at[pl.ds(0, _BB)],
                                  tbl_hbm.at[pl.ds(0, _BB)],
                                  out_sems[k]).wait()

    def interleave(k):
        for g in range(_TSTAGE // _L):
            gb = 64 * g
            for c in range(3):
                val = stages[k][c][pl.ds(_L * g, _L)]
                plsc.store_scatter(ivs[k], [lane4 + (gb + c)], val)

    for p in range(_NBUF - 1):
        fire_stage(p, p)

    def superstep(ss, carry):
        for k in range(_NBUF):
            j = ss * _NBUF + k
            kn = (k + _NBUF - 1) % _NBUF
            if k == 0:
                fire_stage(j + _NBUF - 1, kn)
            else:
                @pl.when(ss < _BSTEPS // _NBUF - 1)
                def _():
                    fire_stage(j + _NBUF - 1, kn)
            wait_stage(k)

            @pl.when(ss >= 1)
            def _():
                wait_out(k)

            interleave(k)
            fire_out(j, k)
        return carry

    lax.fori_loop(0, _BSTEPS // _NBUF, superstep, 0)
    for k in range(_NBUF):
        wait_out(k)


def _gather_body(u_hbm, v_hbm, tbl_hbm, r_hbm, g_hbm, b_hbm,
                 u_a, v_a, u_b, v_b,
                 rA_a, rB_a, rA_b, rB_b,
                 cb_a, cbd_a, cb_b, cbd_b, a_a, b_a, a_b, b_b,
                 cA_a, cB_a, cA_b, cB_b,
                 or_a, og_a, ob_a, or_b, og_b, ob_b,
                 in_a, in_b, g_a, g_b, out_a, out_b):
    wid = lax.axis_index("s") * _NC + lax.axis_index("c")
    base = wid * _PER_W
    lane = lax.iota(jnp.int32, _L)
    uv_bufs = ((u_a, v_a), (u_b, v_b))
    r_bufs = ((rA_a, rB_a), (rA_b, rB_b))
    col_bufs = ((cb_a, cbd_a), (cb_b, cbd_b))
    w_bufs = ((a_a, b_a), (a_b, b_b))
    c_bufs = ((cA_a, cB_a), (cA_b, cB_b))
    o_bufs = ((or_a, og_a, ob_a), (or_b, og_b, ob_b))
    in_sems = (in_a, in_b)
    g_sems = (g_a, g_b)
    out_sems = (out_a, out_b)
    out_hbms = (r_hbm, g_hbm, b_hbm)

    def fire_in(off, k):
        pltpu.async_copy(u_hbm.at[pl.ds(off, _SCH)], uv_bufs[k][0],
                         in_sems[k])
        pltpu.async_copy(v_hbm.at[pl.ds(off, _SCH)], uv_bufs[k][1],
                         in_sems[k])

    def wait_in(k):
        for c in range(2):
            pltpu.make_async_copy(u_hbm.at[pl.ds(0, _SCH)],
                                  uv_bufs[k][c], in_sems[k]).wait()

    def index_phase(k):
        def body(i2, carry):
            for t in range(2):
                i = i2 * 2 + t
                sl = pl.ds(pl.multiple_of(_L * i, _L), _L)
                us = uv_bufs[k][0][sl]
                vs = uv_bufs[k][1][sl]
                u = ((us + 1.0) * 0.5) * (_WIDTH - 1)
                v = ((vs + 1.0) * 0.5) * (_HEIGHT - 1)
                u0 = u.astype(jnp.int32)
                v0 = v.astype(jnp.int32)
                af = u - u0.astype(jnp.float32)
                bf = v - v0.astype(jnp.float32)
                u1 = u0 + jnp.where(af > 0.0, 1, 0)
                dv = jnp.where(bf > 0.0, 1, 0)
                s00 = u0 * _WIDTH + v0
                s10 = u1 * _WIDTH + v0
                w0 = s00 >> 1
                w1 = s10 >> 1
                cb = (s00 & 1) * 4
                r_bufs[k][0][sl] = (w0 >> 1) + ((w0 & 1) << 20)
                r_bufs[k][1][sl] = (w1 >> 1) + ((w1 & 1) << 20)
                col_bufs[k][0][sl] = cb
                col_bufs[k][1][sl] = cb + dv * 4
                w_bufs[k][0][sl] = af
                w_bufs[k][1][sl] = bf
            return carry

        lax.fori_loop(0, _SCH // _L // 2, body, 0)

    def fire_gathers(k):
        for c in range(2):
            for q in range(_QS):
                pltpu.async_copy(
                    tbl_hbm.at[r_bufs[k][c].at[pl.ds(_CHUNK * q, _CHUNK)]],
                    c_bufs[k][c].at[pl.ds(_CHUNK * q, _CHUNK)], g_sems[k])

    def wait_gathers(k):
        for c in range(2):
            for q in range(_QS):
                pltpu.make_async_copy(
                    tbl_hbm.at[r_bufs[k][0].at[pl.ds(0, _CHUNK)]],
                    c_bufs[k][c].at[pl.ds(_CHUNK * q, _CHUNK)],
                    g_sems[k]).wait()

    def combine(k):
        cA_v, cB_v = c_bufs[k]

        def body(i2, carry):
            for t in range(2):
                i = i2 * 2 + t
                sl = pl.ds(pl.multiple_of(_L * i, _L), _L)
                af = w_bufs[k][0][sl]
                bf = w_bufs[k][1][sl]
                cb = col_bufs[k][0][sl]
                cbd = col_bufs[k][1][sl]
                naf = 1.0 - af
                nbf = 1.0 - bf
                row16 = lane + (_L * i)
                for ch in range(3):
                    c00 = plsc.load_gather(cA_v, [row16, cb + ch])
                    c01 = plsc.load_gather(cA_v, [row16, cbd + ch])
                    c10 = plsc.load_gather(cB_v, [row16, cb + ch])
                    c11 = plsc.load_gather(cB_v, [row16, cbd + ch])
                    x = ((c00 * af + c10 * naf) * bf
                         + (c01 * af + c11 * naf) * nbf)
                    xc = jnp.minimum(jnp.maximum(x, -9.0), 9.0)
                    e = jnp.exp(xc + xc)
                    tt = (e - 1.0) / (e + 1.0)
                    o_bufs[k][ch][sl] = tt
            return carry

        lax.fori_loop(0, _SCH // _L // 2, body, 0)

    def fire_outs(off, k):
        for ch in range(3):
            pltpu.async_copy(o_bufs[k][ch],
                             out_hbms[ch].at[pl.ds(off, _SCH)], out_sems[k])

    def wait_outs(k):
        for ch in range(3):
            pltpu.make_async_copy(o_bufs[k][ch],
                                  out_hbms[ch].at[pl.ds(0, _SCH)],
                                  out_sems[k]).wait()

    fire_in(pl.multiple_of(base, _SCH), 0)

    def superstep(ss, carry):
        for k in range(2):
            j = ss * 2 + k
            off = pl.multiple_of(base + j * _SCH, _SCH)
            wait_in(k)
            if k == 0:
                fire_in(off + _SCH, 1)
            else:
                @pl.when(ss < _ITERS // 2 - 1)
                def _():
                    fire_in(off + _SCH, 0)
            index_phase(k)
            fire_gathers(k)

            if k == 0:
                @pl.when(ss >= 1)
                def _():
                    wait_gathers(1)

                    @pl.when(ss >= 2)
                    def _():
                        wait_outs(1)

                    combine(1)
                    fire_outs(off - _SCH, 1)
            else:
                wait_gathers(0)

                @pl.when(ss >= 1)
                def _():
                    wait_outs(0)

                combine(0)
                fire_outs(off - _SCH, 0)
        return carry

    lax.fori_loop(0, _ITERS // 2, superstep, 0)

    last = pl.multiple_of(base + (_ITERS - 1) * _SCH, _SCH)
    wait_gathers(1)
    wait_outs(1)
    combine(1)
    fire_outs(last, 1)
    wait_outs(0)
    wait_outs(1)


_MESH = dict(core_axis_name="c", subcore_axis_name="s",
             num_cores=_NC, num_subcores=_NS)
_CPARAMS = pltpu.CompilerParams(
    needs_layout_passes=False, use_tc_tiling_on_sc=False)


def kernel(uvs, texture):
    u = uvs[:, 0]
    v = uvs[:, 1]
    planes = [jnp.pad(texture[:, :, c].reshape(-1), (0, 16))
              for c in range(3)]

    build = pl.kernel(
        _build_body,
        out_type=jax.ShapeDtypeStruct((2 * _HALF,), jnp.float32),
        mesh=plsc.VectorSubcoreMesh(**_MESH),
        scratch_types=(
            [pltpu.VMEM((_TSTAGE,), jnp.float32)] * (3 * _NBUF)
            + [pltpu.VMEM((_IVLEN,), jnp.float32)] * _NBUF
            + [pltpu.SemaphoreType.DMA] * (2 * _NBUF)),
        compiler_params=_CPARAMS,
    )
    tbl = build(*planes).reshape(2 * _NWIN, 16)

    gather = pl.kernel(
        _gather_body,
        out_type=(jax.ShapeDtypeStruct((_N_UVS,), jnp.float32),) * 3,
        mesh=plsc.VectorSubcoreMesh(**_MESH),
        scratch_types=(
            [pltpu.VMEM((_SCH,), jnp.float32)] * 4
            + [pltpu.VMEM((_SCH,), jnp.int32)] * 4
            + [pltpu.VMEM((_SCH,), jnp.int32)] * 4
            + [pltpu.VMEM((_SCH,), jnp.float32)] * 4
            + [pltpu.VMEM((_SCH, 16), jnp.float32)] * 4
            + [pltpu.VMEM((_SCH,), jnp.float32)] * 6
            + [pltpu.SemaphoreType.DMA] * 6),
        compiler_params=_CPARAMS,
    )
    r, g, b = gather(u, v, tbl)
    return jnp.stack([r, g, b], axis=1)
